# Initial kernel scaffold; baseline (speedup 1.0000x reference)
import jax
import jax.numpy as jnp
from jax.experimental import pallas as pl


def kernel(x, W, b):
    raise NotImplementedError("write your pallas kernel here")


# speedup vs baseline: 32.0683x; 32.0683x over previous
import functools

import jax
import jax.numpy as jnp
import numpy as np
from jax.experimental import pallas as pl
from jax.experimental.pallas import tpu as pltpu

_S = 10
_NCLS = 3
_D = 96
_HW = _D * _D
_K = 20
_EPS = 1e-8
_NEG = -jnp.inf
_BIG = 1e9


def _keep_frac_const():
    key = jax.random.key(42)
    cnt = jnp.zeros((1, 1, _D, _D, _D), jnp.float32)
    for s in range(_S):
        ks = jax.random.fold_in(key, s)
        cnt = cnt + jax.random.bernoulli(ks, 0.5, (1, 1, _D, _D, _D)).astype(
            jnp.float32)
    return np.asarray(cnt).reshape(_D, _D, _D) / np.float32(_S)


_KEEP_FRAC = _keep_frac_const()


def _shift(t, axis, off):
    n = t.shape[axis]
    if off == 0:
        return t
    fill_shape = list(t.shape)
    fill_shape[axis] = abs(off)
    fill = jnp.full(fill_shape, _NEG, t.dtype)
    if off > 0:
        body = jax.lax.slice_in_dim(t, off, n, axis=axis)
        return jnp.concatenate([body, fill], axis=axis)
    body = jax.lax.slice_in_dim(t, 0, n + off, axis=axis)
    return jnp.concatenate([fill, body], axis=axis)


def _pool9_axis(t, axis):
    f2 = jnp.maximum(t, _shift(t, axis, 1))
    f4 = jnp.maximum(f2, _shift(f2, axis, 2))
    r = jnp.maximum(f4, _shift(t, axis, 4))
    g2 = jnp.maximum(_shift(t, axis, -1), _shift(t, axis, -2))
    l = jnp.maximum(g2, _shift(g2, axis, -2))
    return jnp.maximum(l, r)


def _pool9(t):
    t = _pool9_axis(t, 2)
    t = _pool9_axis(t, 1)
    return _pool9_axis(t, 0)


def _body(scal_ref, x_ref, a_ref, co_ref):
    x = x_ref[...]
    a = a_ref[...]
    w2 = [scal_ref[0], scal_ref[1], scal_ref[2]]
    bb = [scal_ref[3], scal_ref[4], scal_ref[5]]
    p0 = [scal_ref[6], scal_ref[7], scal_ref[8]]
    h0 = scal_ref[9]

    z = [x * w2[k] + bb[k] for k in range(_NCLS)]
    m = jnp.maximum(jnp.maximum(z[0], z[1]), z[2])
    e = [jnp.exp(z[k] - m) for k in range(_NCLS)]
    inv = 1.0 / (e[0] + e[1] + e[2])
    p1 = [e[k] * inv for k in range(_NCLS)]

    h1 = -(p1[0] * jnp.log(p1[0] + _EPS)
           + p1[1] * jnp.log(p1[1] + _EPS)
           + p1[2] * jnp.log(p1[2] + _EPS))
    h_exp = (1.0 - a) * h0 + a * h1

    mp = [(1.0 - a) * p0[k] + a * p1[k] for k in range(_NCLS)]
    h_pred = -(mp[0] * jnp.log(mp[0] + _EPS)
               + mp[1] * jnp.log(mp[1] + _EPS)
               + mp[2] * jnp.log(mp[2] + _EPS))
    h_epi = h_pred - h_exp

    en = (h_epi - jnp.min(h_epi)) / (jnp.max(h_epi) - jnp.min(h_epi) + _EPS)
    an = (h_exp - jnp.min(h_exp)) / (jnp.max(h_exp) - jnp.min(h_exp) + _EPS)
    score = 0.5 * (1.0 - en) + 0.5 * (1.0 - an)

    di = jax.lax.broadcasted_iota(jnp.int32, (_D, _D, _D), 0)
    hi = jax.lax.broadcasted_iota(jnp.int32, (_D, _D, _D), 1)
    wi = jax.lax.broadcasted_iota(jnp.int32, (_D, _D, _D), 2)
    flat_iota = (di * _HW + hi * _D + wi).astype(jnp.float32)

    sel_lane = jax.lax.broadcasted_iota(jnp.int32, (8, 128), 1)
    sel_row = jax.lax.broadcasted_iota(jnp.int32, (8, 128), 0)

    for c in range(_NCLS):
        css = score * mp[c]
        pooled = _pool9(css)
        vals = jnp.where(css == pooled, css, 0.0)

        def topk_step(k, carry, _fi=flat_iota):
            v, sel = carry
            mval = jnp.max(v)
            fidx = jnp.min(jnp.where(v == mval, _fi, _BIG))
            v = jnp.where(_fi == fidx, _NEG, v)
            sel = jnp.where((sel_row == 0) & (sel_lane == k), fidx, sel)
            return v, sel

        sel0 = jnp.zeros((8, 128), jnp.float32)
        _, sel = jax.lax.fori_loop(0, _K, topk_step, (vals, sel0))

        d = jnp.floor(sel * (1.0 / _HW))
        rem = sel - d * float(_HW)
        h = jnp.floor(rem * (1.0 / _D))
        w = rem - h * float(_D)
        co_ref[c, 0:1, :] = d[0:1, :]
        co_ref[c, 1:2, :] = h[0:1, :]
        co_ref[c, 2:3, :] = w[0:1, :]


@jax.jit
def kernel(x, W, b):
    x3 = x.reshape(_D, _D, _D)
    a3 = jnp.asarray(_KEEP_FRAC)

    w2 = 2.0 * W.reshape(_NCLS)
    bv = b.reshape(_NCLS)
    p0 = jax.nn.softmax(bv)
    h0 = -jnp.sum(p0 * jnp.log(p0 + _EPS))
    scal = jnp.concatenate([w2, bv, p0, h0[None], jnp.zeros(6, jnp.float32)])

    co = pl.pallas_call(
        _body,
        out_shape=jax.ShapeDtypeStruct((_NCLS, 8, 128), jnp.float32),
        in_specs=[
            pl.BlockSpec(memory_space=pltpu.SMEM),
            pl.BlockSpec(memory_space=pltpu.VMEM),
            pl.BlockSpec(memory_space=pltpu.VMEM),
        ],
        out_specs=pl.BlockSpec(memory_space=pltpu.VMEM),
    )(scal, x3, a3)

    coords = jnp.transpose(co[:, :3, :_K], (0, 2, 1)).reshape(1, _NCLS * _K, 3)
    labels = jnp.broadcast_to(
        jnp.arange(_NCLS).reshape(1, _NCLS, 1), (1, _NCLS, _K)
    ).reshape(1, _NCLS * _K).astype(jnp.int64)
    return (coords.astype(jnp.float32), labels)


# speedup vs baseline: 72.5835x; 2.2634x over previous
import functools

import jax
import jax.numpy as jnp
import numpy as np
from jax.experimental import pallas as pl
from jax.experimental.pallas import tpu as pltpu

_S = 10
_NCLS = 3
_D = 96
_HW = _D * _D
_K = 20
_EPS = 1e-8
_NEG = -jnp.inf
_BIG = 1e9


_ROT = ((13, 15, 26, 6), (17, 29, 16, 24))


def _threefry2x32(k0, k1, x0, x1):
    k0 = np.uint32(k0)
    k1 = np.uint32(k1)
    ks = (k0, k1, np.uint32(np.uint32(0x1BD11BDA) ^ k0 ^ k1))
    x0 = x0.astype(np.uint32) + ks[0]
    x1 = x1.astype(np.uint32) + ks[1]
    for j in range(1, 6):
        for r in _ROT[(j - 1) % 2]:
            x0 = x0 + x1
            x1 = (x1 << np.uint32(r)) | (x1 >> np.uint32(32 - r))
            x1 = x1 ^ x0
        x0 = x0 + ks[j % 3]
        x1 = x1 + ks[(j + 1) % 3] + np.uint32(j)
    return x0, x1


def _keep_frac_const():
    num = _D * _D * _D
    cnt64 = np.zeros(num, np.int64)
    iota = np.arange(num, dtype=np.uint32)
    zeros = np.zeros(num, np.uint32)
    for s in range(_S):
        f0, f1 = _threefry2x32(0, 42, np.uint32([0]), np.uint32([s]))
        b0, b1 = _threefry2x32(f0[0], f1[0], zeros, iota)
        bits = b0 ^ b1
        u = ((bits >> np.uint32(9)) | np.uint32(0x3F800000)).view(
            np.float32) - np.float32(1.0)
        cnt64 += (u < np.float32(0.5))
    return (cnt64.astype(np.float32) / np.float32(_S)).reshape(_D, _D, _D)


_KEEP_FRAC = _keep_frac_const()


def _shift(t, axis, off):
    n = t.shape[axis]
    if off == 0:
        return t
    fill_shape = list(t.shape)
    fill_shape[axis] = abs(off)
    fill = jnp.full(fill_shape, _NEG, t.dtype)
    if off > 0:
        body = jax.lax.slice_in_dim(t, off, n, axis=axis)
        return jnp.concatenate([body, fill], axis=axis)
    body = jax.lax.slice_in_dim(t, 0, n + off, axis=axis)
    return jnp.concatenate([fill, body], axis=axis)


def _pool9_axis(t, axis):
    f2 = jnp.maximum(t, _shift(t, axis, 1))
    f4 = jnp.maximum(f2, _shift(f2, axis, 2))
    r = jnp.maximum(f4, _shift(t, axis, 4))
    g2 = jnp.maximum(_shift(t, axis, -1), _shift(t, axis, -2))
    l = jnp.maximum(g2, _shift(g2, axis, -2))
    return jnp.maximum(l, r)


def _pool9(t):
    t = _pool9_axis(t, 2)
    t = _pool9_axis(t, 1)
    return _pool9_axis(t, 0)


def _body(scal_ref, x_ref, a_ref, co_ref, vals_ref):
    x = x_ref[...]
    a = a_ref[...]
    w2 = [scal_ref[0], scal_ref[1], scal_ref[2]]
    bb = [scal_ref[3], scal_ref[4], scal_ref[5]]
    p0 = [scal_ref[6], scal_ref[7], scal_ref[8]]
    h0 = scal_ref[9]

    z = [x * w2[k] + bb[k] for k in range(_NCLS)]
    m = jnp.maximum(jnp.maximum(z[0], z[1]), z[2])
    e = [jnp.exp(z[k] - m) for k in range(_NCLS)]
    inv = 1.0 / (e[0] + e[1] + e[2])
    p1 = [e[k] * inv for k in range(_NCLS)]

    h1 = -(p1[0] * jnp.log(p1[0] + _EPS)
           + p1[1] * jnp.log(p1[1] + _EPS)
           + p1[2] * jnp.log(p1[2] + _EPS))
    h_exp = (1.0 - a) * h0 + a * h1

    mp = [(1.0 - a) * p0[k] + a * p1[k] for k in range(_NCLS)]
    h_pred = -(mp[0] * jnp.log(mp[0] + _EPS)
               + mp[1] * jnp.log(mp[1] + _EPS)
               + mp[2] * jnp.log(mp[2] + _EPS))
    h_epi = h_pred - h_exp

    en = (h_epi - jnp.min(h_epi)) / (jnp.max(h_epi) - jnp.min(h_epi) + _EPS)
    an = (h_exp - jnp.min(h_exp)) / (jnp.max(h_exp) - jnp.min(h_exp) + _EPS)
    score = 0.5 * (1.0 - en) + 0.5 * (1.0 - an)

    hi2 = jax.lax.broadcasted_iota(jnp.int32, (_D, _D), 0)
    wi2 = jax.lax.broadcasted_iota(jnp.int32, (_D, _D), 1)
    slab_iota = (hi2 * _D + wi2).astype(jnp.float32)

    sel_lane = jax.lax.broadcasted_iota(jnp.int32, (8, 128), 1)
    sel_row = jax.lax.broadcasted_iota(jnp.int32, (8, 128), 0)

    for c in range(_NCLS):
        css = score * mp[c]
        pooled = _pool9(css)
        vals_ref[...] = jnp.where(css == pooled, css, 0.0)
        row_max = jnp.max(vals_ref[...], axis=2)

        def topk_step(k, carry):
            rm, sel = carry
            mval = jnp.max(rm)
            r = jnp.min(jnp.where(rm == mval, hi2, _D))
            slab = vals_ref[r]
            fid2 = jnp.min(jnp.where(slab == mval, slab_iota, _BIG))
            slab2 = jnp.where(slab_iota == fid2, _NEG, slab)
            vals_ref[r] = slab2
            new_rows = jnp.max(slab2, axis=1, keepdims=True)
            rm = jnp.where(hi2 == r, jnp.broadcast_to(new_rows.T, (_D, _D)),
                           rm)
            fidx = r.astype(jnp.float32) * float(_HW) + fid2
            sel = jnp.where((sel_row == 0) & (sel_lane == k), fidx, sel)
            return rm, sel

        sel0 = jnp.zeros((8, 128), jnp.float32)
        _, sel = jax.lax.fori_loop(0, _K, topk_step, (row_max, sel0))

        d = jnp.floor(sel * (1.0 / _HW))
        rem = sel - d * float(_HW)
        h = jnp.floor(rem * (1.0 / _D))
        w = rem - h * float(_D)
        co_ref[c, 0:1, :] = d[0:1, :]
        co_ref[c, 1:2, :] = h[0:1, :]
        co_ref[c, 2:3, :] = w[0:1, :]


@jax.jit
def kernel(x, W, b):
    x3 = x.reshape(_D, _D, _D)
    a3 = jnp.asarray(_KEEP_FRAC)

    w2 = 2.0 * W.reshape(_NCLS)
    bv = b.reshape(_NCLS)
    p0 = jax.nn.softmax(bv)
    h0 = -jnp.sum(p0 * jnp.log(p0 + _EPS))
    scal = jnp.concatenate([w2, bv, p0, h0[None], jnp.zeros(6, jnp.float32)])

    co = pl.pallas_call(
        _body,
        out_shape=jax.ShapeDtypeStruct((_NCLS, 8, 128), jnp.float32),
        in_specs=[
            pl.BlockSpec(memory_space=pltpu.SMEM),
            pl.BlockSpec(memory_space=pltpu.VMEM),
            pl.BlockSpec(memory_space=pltpu.VMEM),
        ],
        out_specs=pl.BlockSpec(memory_space=pltpu.VMEM),
        scratch_shapes=[pltpu.VMEM((_D, _D, _D), jnp.float32)],
    )(scal, x3, a3)

    coords = jnp.transpose(co[:, :3, :_K], (0, 2, 1)).reshape(1, _NCLS * _K, 3)
    labels = jnp.broadcast_to(
        jnp.arange(_NCLS).reshape(1, _NCLS, 1), (1, _NCLS, _K)
    ).reshape(1, _NCLS * _K).astype(jnp.int64)
    return (coords.astype(jnp.float32), labels)
